# baseline (speedup 1.0000x reference)
import functools

import jax
import jax.numpy as jnp
from jax import lax
from jax.experimental import pallas as pl
from jax.experimental.pallas import tpu as pltpu
from jax.experimental.pallas import tpu_sc as plsc

N = 10000
E = 320000
D = 128
DE = 16
DQ = D + DE

NC = 2
NS = 16
NW = NC * NS
EW = E // NW
B = 80
STEPS = EW // B
TS = STEPS
CH = 5
CE = CH * B
STRIPE = N // NS
INV_SQRT_C = 1.0 / (128.0 ** 0.5)

_MESH = plsc.VectorSubcoreMesh(core_axis_name="c", subcore_axis_name="s")
_SC_PARAMS = pltpu.CompilerParams(needs_layout_passes=False,
                                  use_tc_tiling_on_sc=False)


def _project(xb, wq, bq, wet):
  q = (jnp.dot(xb, wq[...], preferred_element_type=jnp.float32)
       + bq[...]) * INV_SQRT_C
  qe = jnp.dot(q, wet[...], preferred_element_type=jnp.float32)
  return jnp.concatenate([q, qe], axis=1)


def _dense_body(x_ref, wq, bq, wk, bk, wv, bv, wet, ws, bs,
                qq_o, k_o, v_o, s_o):
  xb = x_ref[...]
  qq_o[...] = _project(xb, wq, bq, wet)
  k_o[...] = jnp.dot(xb, wk[...], preferred_element_type=jnp.float32) + bk[...]
  v_o[...] = jnp.dot(xb, wv[...], preferred_element_type=jnp.float32) + bv[...]
  s_o[...] = jnp.dot(xb, ws[...], preferred_element_type=jnp.float32) + bs[...]


def _combine(a0, a1, e0, e1, sk, we):
  es = e0[...] + e1[...]
  eagg = es[:, :DE]
  den = es[:, DE:DE + 1]
  num = a0[...] + a1[...] + jnp.dot(eagg, we[...],
                                    preferred_element_type=jnp.float32)
  agg = jnp.where(den > 0, num / den, 0.0)
  return agg + sk[...]


def _mid_body(a0, a1, e0, e1, sk, we,
              wq, bq, wk, bk, wv, bv, wet, ws, bs,
              qq_o, k_o, v_o, s_o):
  h = jnp.maximum(_combine(a0, a1, e0, e1, sk, we), 0.0)
  qq_o[...] = _project(h, wq, bq, wet)
  k_o[...] = jnp.dot(h, wk[...], preferred_element_type=jnp.float32) + bk[...]
  v_o[...] = jnp.dot(h, wv[...], preferred_element_type=jnp.float32) + bv[...]
  s_o[...] = jnp.dot(h, ws[...], preferred_element_type=jnp.float32) + bs[...]


def _fin_body(a0, a1, e0, e1, sk, we, out):
  out[...] = _combine(a0, a1, e0, e1, sk, we)


_R = 1000
_G = N // _R


def _row_spec(w):
  return pl.BlockSpec((_R, w), lambda i: (i, 0))


def _full_spec(shape):
  return pl.BlockSpec(shape, lambda i: tuple(0 for _ in shape))


_W128 = _full_spec((D, D))
_B128 = _full_spec((1, D))
_WET = _full_spec((D, DE))
_WE = _full_spec((DE, D))

_DENSE_OUT = [
    jax.ShapeDtypeStruct((N, DQ), jnp.float32),
    jax.ShapeDtypeStruct((N, D), jnp.float32),
    jax.ShapeDtypeStruct((N, D), jnp.float32),
    jax.ShapeDtypeStruct((N, D), jnp.float32),
]
_DENSE_OUT_SPECS = [_row_spec(DQ)] + [_row_spec(D)] * 3


def _tc_dense(x, wq, bq, wk, bk, wv, bv, wet, ws, bs):
  return pl.pallas_call(
      _dense_body,
      grid=(_G,),
      in_specs=[_row_spec(D), _W128, _B128, _W128, _B128, _W128, _B128,
                _WET, _W128, _B128],
      out_specs=_DENSE_OUT_SPECS,
      out_shape=_DENSE_OUT,
  )(x, wq, bq, wk, bk, wv, bv, wet, ws, bs)


def _tc_mid(a0, a1, e0, e1, sk, we, wq, bq, wk, bk, wv, bv, wet, ws, bs):
  return pl.pallas_call(
      _mid_body,
      grid=(_G,),
      in_specs=[_row_spec(D), _row_spec(D), _row_spec(2 * DE),
                _row_spec(2 * DE), _row_spec(D), _WE,
                _W128, _B128, _W128, _B128, _W128, _B128, _WET, _W128, _B128],
      out_specs=_DENSE_OUT_SPECS,
      out_shape=_DENSE_OUT,
  )(a0, a1, e0, e1, sk, we, wq, bq, wk, bk, wv, bv, wet, ws, bs)


def _tc_fin(a0, a1, e0, e1, sk, we):
  return pl.pallas_call(
      _fin_body,
      grid=(_G,),
      in_specs=[_row_spec(D), _row_spec(D), _row_spec(2 * DE),
                _row_spec(2 * DE), _row_spec(D), _WE],
      out_specs=_row_spec(D),
      out_shape=jax.ShapeDtypeStruct((N, D), jnp.float32),
  )(a0, a1, e0, e1, sk, we)


@functools.partial(
    pl.kernel,
    out_type=[
        jax.ShapeDtypeStruct((E,), jnp.float32),
        jax.ShapeDtypeStruct((NW * 16,), jnp.float32),
    ],
    mesh=_MESH,
    compiler_params=_SC_PARAMS,
    scratch_types=[
        pltpu.VMEM((TS, B), jnp.int32),
        pltpu.VMEM((TS, B), jnp.int32),
        pltpu.VMEM((CE, DE), jnp.float32),
        pltpu.VMEM((EW,), jnp.float32),
        pltpu.VMEM((B, DQ), jnp.float32),
        pltpu.VMEM((B, DQ), jnp.float32),
        pltpu.VMEM((B, D), jnp.float32),
        pltpu.VMEM((B, D), jnp.float32),
        pltpu.VMEM((16,), jnp.float32),
        pltpu.SemaphoreType.DMA,
        pltpu.SemaphoreType.DMA,
    ],
)
def _sc_pass_a(srcw_h, dstw_h, ea_h, qq_h, kt_h, alpha_h, tmax_h,
               srcw_c, dstw_c, ea_c, al_c, qq0, qq1, k0, k1, mx_v,
               g0, g1):
  c = lax.axis_index("c")
  s = lax.axis_index("s")
  wid = c * NS + s
  base = wid * EW
  trow = wid * TS
  lanes = lax.broadcasted_iota(jnp.int32, (16,), 0)

  pltpu.sync_copy(srcw_h.at[pl.ds(trow, TS)], srcw_c)
  pltpu.sync_copy(dstw_h.at[pl.ds(trow, TS)], dstw_c)
  pltpu.sync_copy(ea_h.at[pl.ds(base, CE)], ea_c)

  def fire(j, qq_b, k_b, sem):
    pltpu.make_async_copy(qq_h.at[dstw_c.at[j]], qq_b, sem).start()
    pltpu.make_async_copy(kt_h.at[srcw_c.at[j]], k_b, sem).start()

  def wait(qq_b, k_b, sem):
    pltpu.make_async_copy(qq_h.at[dstw_c.at[0]], qq_b, sem).wait()
    pltpu.make_async_copy(kt_h.at[srcw_c.at[0]], k_b, sem).wait()

  def load_chunk(j):
    pltpu.sync_copy(ea_h.at[pl.ds(base + (j // CH) * CE, CE)], ea_c)

  rotp = [(lanes + h) % 16 for h in (8, 4, 2, 1)]
  rotm = [(lanes - h) % 16 for h in (8, 4, 2, 1)]
  low = [((lanes // h) % 2) == 0 for h in (8, 4, 2, 1)]
  bitrev = (((lanes & 1) << 3) | ((lanes & 2) << 1)
            | ((lanes & 4) >> 1) | ((lanes & 8) >> 3))

  def _rot(v, t):
    return jnp.take_along_axis(v, t, axis=0, mode="promise_in_bounds")

  def _comb(x, y, hi):
    return jnp.where(low[hi], x + _rot(x, rotp[hi]), y + _rot(y, rotm[hi]))

  def compute(j, qq_b, k_b, mx):
    jr = (j % CH) * B
    for g in range(B // 16):
      vecs = []
      for t in range(16):
        b = g * 16 + t
        p = [qq_b[b, pl.ds(u * 16, 16)] * k_b[b, pl.ds(u * 16, 16)]
             for u in range(D // 16)]
        p.append(qq_b[b, pl.ds(D, DE)] * ea_c[jr + b, :])
        vecs.append(((p[0] + p[1]) + (p[2] + p[3]))
                    + ((p[4] + p[5]) + (p[6] + p[7]) + p[8]))
      for hi in range(4):
        vecs = [_comb(vecs[2 * i], vecs[2 * i + 1], hi)
                for i in range(len(vecs) // 2)]
      av = _rot(vecs[0], bitrev)
      al_c[pl.ds(j * B + g * 16, 16)] = av
      mx = jnp.maximum(mx, av)
    return mx

  fire(0, qq0, k0, g0)

  def body(k, mx):
    s0 = 2 * k
    s1 = 2 * k + 1
    fire(s1, qq1, k1, g1)

    @pl.when((s0 > 0) & (s0 % CH == 0))
    def _():
      load_chunk(s0)

    wait(qq0, k0, g0)
    mx = compute(s0, qq0, k0, mx)
    fire(s0 + 2, qq0, k0, g0)

    @pl.when(s1 % CH == 0)
    def _():
      load_chunk(s1)

    wait(qq1, k1, g1)
    mx = compute(s1, qq1, k1, mx)
    return mx

  mx = lax.fori_loop(0, (STEPS - 1) // 2, body,
                     jnp.full((16,), -jnp.inf, jnp.float32))
  wait(qq0, k0, g0)
  mx = compute(STEPS - 1, qq0, k0, mx)

  mx_v[...] = mx
  pltpu.sync_copy(mx_v, tmax_h.at[pl.ds(wid * 16, 16)])
  pltpu.sync_copy(al_c, alpha_h.at[pl.ds(base, EW)])


CHB = 25
CEB = CHB * B

@functools.partial(
    pl.kernel,
    out_type=[
        jax.ShapeDtypeStruct((N, D), jnp.float32),
        jax.ShapeDtypeStruct((N, D), jnp.float32),
    ],
    mesh=_MESH,
    compiler_params=_SC_PARAMS,
    scratch_types=[
        pltpu.VMEM((TS, B), jnp.int32),
        pltpu.VMEM((TS, B), jnp.int32),
        pltpu.VMEM((CEB,), jnp.float32),
        pltpu.VMEM((B, D), jnp.float32),
        pltpu.VMEM((B, D), jnp.float32),
        pltpu.VMEM((NW * 16,), jnp.float32),
        pltpu.VMEM_SHARED((N, D), jnp.float32),
        pltpu.SemaphoreType.DMA,
        pltpu.SemaphoreType.DMA,
        pltpu.SemaphoreType.DMA,
        pltpu.SemaphoreType.DMA,
    ],
)
def _sc_pass_b(srcw_h, dstw_h, alpha_h, tmax_h, vt_h, zagg_h,
               agg0_h, agg1_h,
               srcw_c, dstw_c, al_c, v0, v1,
               mxb_v, aggv_sh, gv0, gv1, sc0, sc1):
  c = lax.axis_index("c")
  s = lax.axis_index("s")
  wid = c * NS + s
  base = wid * EW
  trow = wid * TS
  row0 = s * STRIPE

  pltpu.sync_copy(zagg_h.at[pl.ds(row0, STRIPE)],
                  aggv_sh.at[pl.ds(row0, STRIPE)])
  plsc.subcore_barrier()

  pltpu.sync_copy(tmax_h, mxb_v)
  acc = mxb_v[pl.ds(0, 16)]
  for w in range(1, NW):
    acc = jnp.maximum(acc, mxb_v[pl.ds(w * 16, 16)])
  gmax = jnp.max(acc)

  pltpu.sync_copy(srcw_h.at[pl.ds(trow, TS)], srcw_c)
  pltpu.sync_copy(dstw_h.at[pl.ds(trow, TS)], dstw_c)
  pltpu.sync_copy(alpha_h.at[pl.ds(base, CEB)], al_c)

  def fire_v(j, v_b, sem):
    pltpu.make_async_copy(vt_h.at[srcw_c.at[j]], v_b, sem).start()

  def wait_v(v_b, sem):
    pltpu.make_async_copy(vt_h.at[srcw_c.at[0]], v_b, sem).wait()

  def fire_sc(j, v_b, sem):
    pltpu.async_copy(v_b, aggv_sh.at[dstw_c.at[j]], sem, add=True)

  def wait_sc(v_b, sem):
    pltpu.make_async_copy(v_b, aggv_sh.at[dstw_c.at[0]], sem).wait()

  def load_chunk(j):
    pltpu.sync_copy(alpha_h.at[pl.ds(base + (j // CHB) * CEB, CEB)], al_c)

  def compute(j, v_b):
    jr = (j % CHB) * B
    for g in range(B // 16):
      exg = jnp.exp(al_c[pl.ds(jr + g * 16, 16)] - gmax)
      for t in range(16):
        b = g * 16 + t
        sx = jnp.take_along_axis(exg, jnp.full((16,), t, jnp.int32),
                                 axis=0, mode="promise_in_bounds")
        for u in range(D // 16):
          v_b[b, pl.ds(u * 16, 16)] = v_b[b, pl.ds(u * 16, 16)] * sx

  fire_v(0, v0, gv0)
  fire_v(1, v1, gv1)

  def body(k, carry):
    s0 = 2 * k
    s1 = 2 * k + 1

    @pl.when((s0 > 0) & (s0 % CHB == 0))
    def _():
      load_chunk(s0)

    wait_v(v0, gv0)
    compute(s0, v0)
    fire_sc(s0, v0, sc0)
    wait_sc(v0, sc0)
    fire_v(jnp.minimum(s0 + 2, STEPS - 1), v0, gv0)

    @pl.when(s1 % CHB == 0)
    def _():
      load_chunk(s1)

    wait_v(v1, gv1)
    compute(s1, v1)
    fire_sc(s1, v1, sc1)
    wait_sc(v1, sc1)
    fire_v(jnp.minimum(s1 + 2, STEPS - 1), v1, gv1)
    return carry

  lax.fori_loop(0, (STEPS - 1) // 2, body, 0)

  wait_v(v0, gv0)
  compute(STEPS - 1, v0)
  fire_sc(STEPS - 1, v0, sc0)
  wait_sc(v0, sc0)
  wait_v(v1, gv1)

  plsc.subcore_barrier()

  @pl.when(c == 0)
  def _():
    pltpu.sync_copy(aggv_sh.at[pl.ds(row0, STRIPE)],
                    agg0_h.at[pl.ds(row0, STRIPE)])

  @pl.when(c == 1)
  def _():
    pltpu.sync_copy(aggv_sh.at[pl.ds(row0, STRIPE)],
                    agg1_h.at[pl.ds(row0, STRIPE)])


@functools.partial(
    pl.kernel,
    out_type=[
        jax.ShapeDtypeStruct((N, 2 * DE), jnp.float32),
        jax.ShapeDtypeStruct((N, 2 * DE), jnp.float32),
    ],
    mesh=_MESH,
    compiler_params=_SC_PARAMS,
    scratch_types=[
        pltpu.VMEM((TS, B), jnp.int32),
        pltpu.VMEM((CE, DE), jnp.float32),
        pltpu.VMEM((EW,), jnp.float32),
        pltpu.VMEM((B, 2 * DE), jnp.float32),
        pltpu.VMEM((B, 2 * DE), jnp.float32),
        pltpu.VMEM((NW * 16,), jnp.float32),
        pltpu.VMEM_SHARED((N, 2 * DE), jnp.float32),
        pltpu.SemaphoreType.DMA,
        pltpu.SemaphoreType.DMA,
    ],
)
def _sc_pass_c(dstw_h, ea_h, alpha_h, tmax_h, zea_h,
               eagg0_h, eagg1_h,
               dstw_c, ea_c, al_c, ec0, ec1, mxb_v, eagg_sh, sc0, sc1):
  c = lax.axis_index("c")
  s = lax.axis_index("s")
  wid = c * NS + s
  base = wid * EW
  trow = wid * TS
  lanes = lax.broadcasted_iota(jnp.int32, (16,), 0)
  row0 = s * STRIPE

  pltpu.sync_copy(zea_h.at[pl.ds(row0, STRIPE)],
                  eagg_sh.at[pl.ds(row0, STRIPE)])
  plsc.subcore_barrier()

  pltpu.sync_copy(tmax_h, mxb_v)
  acc = mxb_v[pl.ds(0, 16)]
  for w in range(1, NW):
    acc = jnp.maximum(acc, mxb_v[pl.ds(w * 16, 16)])
  gmax = jnp.max(acc)

  pltpu.sync_copy(dstw_h.at[pl.ds(trow, TS)], dstw_c)
  pltpu.sync_copy(alpha_h.at[pl.ds(base, EW)], al_c)
  pltpu.sync_copy(ea_h.at[pl.ds(base, CE)], ea_c)

  def fire_sc(j, ec, sem):
    pltpu.async_copy(ec, eagg_sh.at[dstw_c.at[j]], sem, add=True)

  def wait_sc(ec, sem):
    pltpu.make_async_copy(ec, eagg_sh.at[dstw_c.at[0]], sem).wait()

  def load_chunk(j):
    pltpu.sync_copy(ea_h.at[pl.ds(base + (j // CH) * CE, CE)], ea_c)

  def compute(j, ec):
    jr = (j % CH) * B
    for g in range(B // 16):
      exg = jnp.exp(al_c[pl.ds(j * B + g * 16, 16)] - gmax)
      for t in range(16):
        b = g * 16 + t
        sx = jnp.take_along_axis(exg, jnp.full((16,), t, jnp.int32),
                                 axis=0, mode="promise_in_bounds")
        ec[b, pl.ds(0, DE)] = ea_c[jr + b, :] * sx
        ec[b, pl.ds(DE, DE)] = jnp.where(lanes == 0, sx, 0.0)

  def body(k, carry):
    s0 = 2 * k
    s1 = 2 * k + 1

    @pl.when(s0 > 0)
    def _():
      wait_sc(ec0, sc0)

    @pl.when((s0 > 0) & (s0 % CH == 0))
    def _():
      load_chunk(s0)

    compute(s0, ec0)
    fire_sc(s0, ec0, sc0)

    @pl.when(s1 > 1)
    def _():
      wait_sc(ec1, sc1)

    @pl.when(s1 % CH == 0)
    def _():
      load_chunk(s1)

    compute(s1, ec1)
    fire_sc(s1, ec1, sc1)
    return carry

  lax.fori_loop(0, (STEPS - 1) // 2, body, 0)

  wait_sc(ec0, sc0)
  compute(STEPS - 1, ec0)
  fire_sc(STEPS - 1, ec0, sc0)
  wait_sc(ec0, sc0)
  wait_sc(ec1, sc1)

  plsc.subcore_barrier()

  @pl.when(c == 0)
  def _():
    pltpu.sync_copy(eagg_sh.at[pl.ds(row0, STRIPE)],
                    eagg0_h.at[pl.ds(row0, STRIPE)])

  @pl.when(c == 1)
  def _():
    pltpu.sync_copy(eagg_sh.at[pl.ds(row0, STRIPE)],
                    eagg1_h.at[pl.ds(row0, STRIPE)])


def kernel(x, edge_index, edge_attr,
           W1q, b1q, W1k, b1k, W1v, b1v, W1e, W1s, b1s,
           W2q, b2q, W2k, b2k, W2v, b2v, W2e, W2s, b2s):
  srcw = edge_index[0].reshape(E // B, B)
  dstw = edge_index[1].reshape(E // B, B)
  zagg = jnp.zeros((N, D), jnp.float32)
  zea = jnp.zeros((N, 2 * DE), jnp.float32)

  def layer(qq, kt, vt):
    alpha, tmax = _sc_pass_a(srcw, dstw, edge_attr, qq, kt)
    a0, a1 = _sc_pass_b(srcw, dstw, alpha, tmax, vt, zagg)
    e0, e1 = _sc_pass_c(dstw, edge_attr, alpha, tmax, zea)
    return a0, a1, e0, e1

  r = lambda b: b.reshape(1, D)

  qq, kt, vt, sk1 = _tc_dense(x, W1q, r(b1q), W1k, r(b1k), W1v, r(b1v),
                              W1e.T, W1s, r(b1s))
  a0, a1, e0, e1 = layer(qq, kt, vt)
  qq2, kt2, vt2, sk2 = _tc_mid(a0, a1, e0, e1, sk1, W1e,
                               W2q, r(b2q), W2k, r(b2k), W2v, r(b2v),
                               W2e.T, W2s, r(b2s))
  b0, b1_, f0, f1 = layer(qq2, kt2, vt2)
  return _tc_fin(b0, b1_, f0, f1, sk2, W2e)


# speedup vs baseline: 1.1926x; 1.1926x over previous
import functools

import jax
import jax.numpy as jnp
from jax import lax
from jax.experimental import pallas as pl
from jax.experimental.pallas import tpu as pltpu
from jax.experimental.pallas import tpu_sc as plsc

N = 10000
E = 320000
D = 128
DE = 16
DQ = D + DE

NC = 2
NS = 16
NW = NC * NS
EW = E // NW
B = 80
STEPS = EW // B
TS = STEPS
CH = 5
CE = CH * B
STRIPE = N // NS
INV_SQRT_C = 1.0 / (128.0 ** 0.5)

_MESH = plsc.VectorSubcoreMesh(core_axis_name="c", subcore_axis_name="s")
_SC_PARAMS = pltpu.CompilerParams(needs_layout_passes=False,
                                  use_tc_tiling_on_sc=False)


def _project(xb, wq, bq, wet):
  q = (jnp.dot(xb, wq[...], preferred_element_type=jnp.float32)
       + bq[...]) * INV_SQRT_C
  qe = jnp.dot(q, wet[...], preferred_element_type=jnp.float32)
  return jnp.concatenate([q, qe], axis=1)


def _dense_body(x_ref, wq, bq, wk, bk, wv, bv, wet, ws, bs,
                qq_o, k_o, v_o, s_o):
  xb = x_ref[...]
  qq_o[...] = _project(xb, wq, bq, wet)
  k_o[...] = jnp.dot(xb, wk[...], preferred_element_type=jnp.float32) + bk[...]
  v_o[...] = jnp.dot(xb, wv[...], preferred_element_type=jnp.float32) + bv[...]
  s_o[...] = jnp.dot(xb, ws[...], preferred_element_type=jnp.float32) + bs[...]


def _combine(a0, a1, e0, e1, sk, we):
  es = e0[...] + e1[...]
  eagg = es[:, :DE]
  den = es[:, DE:DE + 1]
  num = a0[...] + a1[...] + jnp.dot(eagg, we[...],
                                    preferred_element_type=jnp.float32)
  agg = jnp.where(den > 0, num / den, 0.0)
  return agg + sk[...]


def _mid_body(a0, a1, e0, e1, sk, we,
              wq, bq, wk, bk, wv, bv, wet, ws, bs,
              qq_o, k_o, v_o, s_o):
  h = jnp.maximum(_combine(a0, a1, e0, e1, sk, we), 0.0)
  qq_o[...] = _project(h, wq, bq, wet)
  k_o[...] = jnp.dot(h, wk[...], preferred_element_type=jnp.float32) + bk[...]
  v_o[...] = jnp.dot(h, wv[...], preferred_element_type=jnp.float32) + bv[...]
  s_o[...] = jnp.dot(h, ws[...], preferred_element_type=jnp.float32) + bs[...]


def _fin_body(a0, a1, e0, e1, sk, we, out):
  out[...] = _combine(a0, a1, e0, e1, sk, we)


_R = 1000
_G = N // _R


def _row_spec(w):
  return pl.BlockSpec((_R, w), lambda i: (i, 0))


def _full_spec(shape):
  return pl.BlockSpec(shape, lambda i: tuple(0 for _ in shape))


_W128 = _full_spec((D, D))
_B128 = _full_spec((1, D))
_WET = _full_spec((D, DE))
_WE = _full_spec((DE, D))

_DENSE_OUT = [
    jax.ShapeDtypeStruct((N, DQ), jnp.float32),
    jax.ShapeDtypeStruct((N, D), jnp.float32),
    jax.ShapeDtypeStruct((N, D), jnp.float32),
    jax.ShapeDtypeStruct((N, D), jnp.float32),
]
_DENSE_OUT_SPECS = [_row_spec(DQ)] + [_row_spec(D)] * 3


def _tc_dense(x, wq, bq, wk, bk, wv, bv, wet, ws, bs):
  return pl.pallas_call(
      _dense_body,
      grid=(_G,),
      in_specs=[_row_spec(D), _W128, _B128, _W128, _B128, _W128, _B128,
                _WET, _W128, _B128],
      out_specs=_DENSE_OUT_SPECS,
      out_shape=_DENSE_OUT,
  )(x, wq, bq, wk, bk, wv, bv, wet, ws, bs)


def _tc_mid(a0, a1, e0, e1, sk, we, wq, bq, wk, bk, wv, bv, wet, ws, bs):
  return pl.pallas_call(
      _mid_body,
      grid=(_G,),
      in_specs=[_row_spec(D), _row_spec(D), _row_spec(2 * DE),
                _row_spec(2 * DE), _row_spec(D), _WE,
                _W128, _B128, _W128, _B128, _W128, _B128, _WET, _W128, _B128],
      out_specs=_DENSE_OUT_SPECS,
      out_shape=_DENSE_OUT,
  )(a0, a1, e0, e1, sk, we, wq, bq, wk, bk, wv, bv, wet, ws, bs)


def _tc_fin(a0, a1, e0, e1, sk, we):
  return pl.pallas_call(
      _fin_body,
      grid=(_G,),
      in_specs=[_row_spec(D), _row_spec(D), _row_spec(2 * DE),
                _row_spec(2 * DE), _row_spec(D), _WE],
      out_specs=_row_spec(D),
      out_shape=jax.ShapeDtypeStruct((N, D), jnp.float32),
  )(a0, a1, e0, e1, sk, we)


@functools.partial(
    pl.kernel,
    out_type=[
        jax.ShapeDtypeStruct((E,), jnp.float32),
        jax.ShapeDtypeStruct((NW * 16,), jnp.float32),
    ],
    mesh=_MESH,
    compiler_params=_SC_PARAMS,
    scratch_types=[
        pltpu.VMEM((TS, B), jnp.int32),
        pltpu.VMEM((TS, B), jnp.int32),
        pltpu.VMEM((CE, DE), jnp.float32),
        pltpu.VMEM((EW,), jnp.float32),
        pltpu.VMEM((B, DQ), jnp.float32),
        pltpu.VMEM((B, DQ), jnp.float32),
        pltpu.VMEM((B, D), jnp.float32),
        pltpu.VMEM((B, D), jnp.float32),
        pltpu.VMEM((16,), jnp.float32),
        pltpu.SemaphoreType.DMA,
        pltpu.SemaphoreType.DMA,
    ],
)
def _sc_pass_a(srcw_h, dstw_h, ea_h, qq_h, kt_h, alpha_h, tmax_h,
               srcw_c, dstw_c, ea_c, al_c, qq0, qq1, k0, k1, mx_v,
               g0, g1):
  c = lax.axis_index("c")
  s = lax.axis_index("s")
  wid = c * NS + s
  base = wid * EW
  trow = wid * TS
  lanes = lax.broadcasted_iota(jnp.int32, (16,), 0)

  pltpu.sync_copy(srcw_h.at[pl.ds(trow, TS)], srcw_c)
  pltpu.sync_copy(dstw_h.at[pl.ds(trow, TS)], dstw_c)
  pltpu.sync_copy(ea_h.at[pl.ds(base, CE)], ea_c)

  def fire(j, qq_b, k_b, sem):
    pltpu.make_async_copy(qq_h.at[dstw_c.at[j]], qq_b, sem).start()
    pltpu.make_async_copy(kt_h.at[srcw_c.at[j]], k_b, sem).start()

  def wait(qq_b, k_b, sem):
    pltpu.make_async_copy(qq_h.at[dstw_c.at[0]], qq_b, sem).wait()
    pltpu.make_async_copy(kt_h.at[srcw_c.at[0]], k_b, sem).wait()

  def load_chunk(j):
    pltpu.sync_copy(ea_h.at[pl.ds(base + (j // CH) * CE, CE)], ea_c)

  rotp = [(lanes + h) % 16 for h in (8, 4, 2, 1)]
  rotm = [(lanes - h) % 16 for h in (8, 4, 2, 1)]
  low = [((lanes // h) % 2) == 0 for h in (8, 4, 2, 1)]
  bitrev = (((lanes & 1) << 3) | ((lanes & 2) << 1)
            | ((lanes & 4) >> 1) | ((lanes & 8) >> 3))

  def _rot(v, t):
    return jnp.take_along_axis(v, t, axis=0, mode="promise_in_bounds")

  def _comb(x, y, hi):
    return jnp.where(low[hi], x + _rot(x, rotp[hi]), y + _rot(y, rotm[hi]))

  def compute(j, qq_b, k_b, mx):
    jr = (j % CH) * B

    def grp(g, mx):
      b0 = g * 16

      def acc(t):
        b = b0 + t
        p = [qq_b[b, pl.ds(u * 16, 16)] * k_b[b, pl.ds(u * 16, 16)]
             for u in range(D // 16)]
        p.append(qq_b[b, pl.ds(D, DE)] * ea_c[jr + b, :])
        return (((p[0] + p[1]) + (p[2] + p[3]))
                + ((p[4] + p[5]) + (p[6] + p[7]) + p[8]))

      def red(lo, size):
        if size == 1:
          return acc(lo)
        half = size // 2
        return _comb(red(lo, half), red(lo + half, half),
                     {2: 0, 4: 1, 8: 2, 16: 3}[size])

      av = _rot(red(0, 16), bitrev)
      al_c[pl.ds(j * B + b0, 16)] = av
      return jnp.maximum(mx, av)

    return lax.fori_loop(0, B // 16, grp, mx)

  fire(0, qq0, k0, g0)

  def body(k, mx):
    s0 = 2 * k
    s1 = 2 * k + 1
    fire(s1, qq1, k1, g1)

    @pl.when((s0 > 0) & (s0 % CH == 0))
    def _():
      load_chunk(s0)

    wait(qq0, k0, g0)
    mx = compute(s0, qq0, k0, mx)
    fire(s0 + 2, qq0, k0, g0)

    @pl.when(s1 % CH == 0)
    def _():
      load_chunk(s1)

    wait(qq1, k1, g1)
    mx = compute(s1, qq1, k1, mx)
    return mx

  mx = lax.fori_loop(0, (STEPS - 1) // 2, body,
                     jnp.full((16,), -jnp.inf, jnp.float32))
  wait(qq0, k0, g0)
  mx = compute(STEPS - 1, qq0, k0, mx)

  mx_v[...] = mx
  pltpu.sync_copy(mx_v, tmax_h.at[pl.ds(wid * 16, 16)])
  pltpu.sync_copy(al_c, alpha_h.at[pl.ds(base, EW)])


CHB = 25
CEB = CHB * B

@functools.partial(
    pl.kernel,
    out_type=[
        jax.ShapeDtypeStruct((N, D), jnp.float32),
        jax.ShapeDtypeStruct((N, D), jnp.float32),
    ],
    mesh=_MESH,
    compiler_params=_SC_PARAMS,
    scratch_types=[
        pltpu.VMEM((TS, B), jnp.int32),
        pltpu.VMEM((TS, B), jnp.int32),
        pltpu.VMEM((CEB,), jnp.float32),
        pltpu.VMEM((B, D), jnp.float32),
        pltpu.VMEM((B, D), jnp.float32),
        pltpu.VMEM((NW * 16,), jnp.float32),
        pltpu.VMEM_SHARED((N, D), jnp.float32),
        pltpu.SemaphoreType.DMA,
        pltpu.SemaphoreType.DMA,
        pltpu.SemaphoreType.DMA,
        pltpu.SemaphoreType.DMA,
    ],
)
def _sc_pass_b(srcw_h, dstw_h, alpha_h, tmax_h, vt_h, zagg_h,
               agg0_h, agg1_h,
               srcw_c, dstw_c, al_c, v0, v1,
               mxb_v, aggv_sh, gv0, gv1, sc0, sc1):
  c = lax.axis_index("c")
  s = lax.axis_index("s")
  wid = c * NS + s
  base = wid * EW
  trow = wid * TS
  row0 = s * STRIPE

  pltpu.sync_copy(zagg_h.at[pl.ds(row0, STRIPE)],
                  aggv_sh.at[pl.ds(row0, STRIPE)])
  plsc.subcore_barrier()

  pltpu.sync_copy(tmax_h, mxb_v)
  acc = mxb_v[pl.ds(0, 16)]
  for w in range(1, NW):
    acc = jnp.maximum(acc, mxb_v[pl.ds(w * 16, 16)])
  gmax = jnp.max(acc)

  pltpu.sync_copy(srcw_h.at[pl.ds(trow, TS)], srcw_c)
  pltpu.sync_copy(dstw_h.at[pl.ds(trow, TS)], dstw_c)
  pltpu.sync_copy(alpha_h.at[pl.ds(base, CEB)], al_c)

  def fire_v(j, v_b, sem):
    pltpu.make_async_copy(vt_h.at[srcw_c.at[j]], v_b, sem).start()

  def wait_v(v_b, sem):
    pltpu.make_async_copy(vt_h.at[srcw_c.at[0]], v_b, sem).wait()

  def fire_sc(j, v_b, sem):
    pltpu.async_copy(v_b, aggv_sh.at[dstw_c.at[j]], sem, add=True)

  def wait_sc(v_b, sem):
    pltpu.make_async_copy(v_b, aggv_sh.at[dstw_c.at[0]], sem).wait()

  def load_chunk(j):
    pltpu.sync_copy(alpha_h.at[pl.ds(base + (j // CHB) * CEB, CEB)], al_c)

  def compute(j, v_b):
    jr = (j % CHB) * B
    for g in range(B // 16):
      exg = jnp.exp(al_c[pl.ds(jr + g * 16, 16)] - gmax)
      for t in range(16):
        b = g * 16 + t
        sx = jnp.take_along_axis(exg, jnp.full((16,), t, jnp.int32),
                                 axis=0, mode="promise_in_bounds")
        for u in range(D // 16):
          v_b[b, pl.ds(u * 16, 16)] = v_b[b, pl.ds(u * 16, 16)] * sx

  fire_v(0, v0, gv0)
  fire_v(1, v1, gv1)

  def body(k, carry):
    s0 = 2 * k
    s1 = 2 * k + 1

    @pl.when((s0 > 0) & (s0 % CHB == 0))
    def _():
      load_chunk(s0)

    wait_v(v0, gv0)
    compute(s0, v0)
    fire_sc(s0, v0, sc0)
    wait_sc(v0, sc0)
    fire_v(jnp.minimum(s0 + 2, STEPS - 1), v0, gv0)

    @pl.when(s1 % CHB == 0)
    def _():
      load_chunk(s1)

    wait_v(v1, gv1)
    compute(s1, v1)
    fire_sc(s1, v1, sc1)
    wait_sc(v1, sc1)
    fire_v(jnp.minimum(s1 + 2, STEPS - 1), v1, gv1)
    return carry

  lax.fori_loop(0, (STEPS - 1) // 2, body, 0)

  wait_v(v0, gv0)
  compute(STEPS - 1, v0)
  fire_sc(STEPS - 1, v0, sc0)
  wait_sc(v0, sc0)
  wait_v(v1, gv1)

  plsc.subcore_barrier()

  @pl.when(c == 0)
  def _():
    pltpu.sync_copy(aggv_sh.at[pl.ds(row0, STRIPE)],
                    agg0_h.at[pl.ds(row0, STRIPE)])

  @pl.when(c == 1)
  def _():
    pltpu.sync_copy(aggv_sh.at[pl.ds(row0, STRIPE)],
                    agg1_h.at[pl.ds(row0, STRIPE)])


@functools.partial(
    pl.kernel,
    out_type=[
        jax.ShapeDtypeStruct((N, 2 * DE), jnp.float32),
        jax.ShapeDtypeStruct((N, 2 * DE), jnp.float32),
    ],
    mesh=_MESH,
    compiler_params=_SC_PARAMS,
    scratch_types=[
        pltpu.VMEM((TS, B), jnp.int32),
        pltpu.VMEM((CE, DE), jnp.float32),
        pltpu.VMEM((EW,), jnp.float32),
        pltpu.VMEM((B, 2 * DE), jnp.float32),
        pltpu.VMEM((B, 2 * DE), jnp.float32),
        pltpu.VMEM((NW * 16,), jnp.float32),
        pltpu.VMEM_SHARED((N, 2 * DE), jnp.float32),
        pltpu.SemaphoreType.DMA,
        pltpu.SemaphoreType.DMA,
    ],
)
def _sc_pass_c(dstw_h, ea_h, alpha_h, tmax_h, zea_h,
               eagg0_h, eagg1_h,
               dstw_c, ea_c, al_c, ec0, ec1, mxb_v, eagg_sh, sc0, sc1):
  c = lax.axis_index("c")
  s = lax.axis_index("s")
  wid = c * NS + s
  base = wid * EW
  trow = wid * TS
  lanes = lax.broadcasted_iota(jnp.int32, (16,), 0)
  row0 = s * STRIPE

  pltpu.sync_copy(zea_h.at[pl.ds(row0, STRIPE)],
                  eagg_sh.at[pl.ds(row0, STRIPE)])
  plsc.subcore_barrier()

  pltpu.sync_copy(tmax_h, mxb_v)
  acc = mxb_v[pl.ds(0, 16)]
  for w in range(1, NW):
    acc = jnp.maximum(acc, mxb_v[pl.ds(w * 16, 16)])
  gmax = jnp.max(acc)

  pltpu.sync_copy(dstw_h.at[pl.ds(trow, TS)], dstw_c)
  pltpu.sync_copy(alpha_h.at[pl.ds(base, EW)], al_c)
  pltpu.sync_copy(ea_h.at[pl.ds(base, CE)], ea_c)

  def fire_sc(j, ec, sem):
    pltpu.async_copy(ec, eagg_sh.at[dstw_c.at[j]], sem, add=True)

  def wait_sc(ec, sem):
    pltpu.make_async_copy(ec, eagg_sh.at[dstw_c.at[0]], sem).wait()

  def load_chunk(j):
    pltpu.sync_copy(ea_h.at[pl.ds(base + (j // CH) * CE, CE)], ea_c)

  def compute(j, ec):
    jr = (j % CH) * B
    for g in range(B // 16):
      exg = jnp.exp(al_c[pl.ds(j * B + g * 16, 16)] - gmax)
      for t in range(16):
        b = g * 16 + t
        sx = jnp.take_along_axis(exg, jnp.full((16,), t, jnp.int32),
                                 axis=0, mode="promise_in_bounds")
        ec[b, pl.ds(0, DE)] = ea_c[jr + b, :] * sx
        ec[b, pl.ds(DE, DE)] = jnp.where(lanes == 0, sx, 0.0)

  def body(k, carry):
    s0 = 2 * k
    s1 = 2 * k + 1

    @pl.when(s0 > 0)
    def _():
      wait_sc(ec0, sc0)

    @pl.when((s0 > 0) & (s0 % CH == 0))
    def _():
      load_chunk(s0)

    compute(s0, ec0)
    fire_sc(s0, ec0, sc0)

    @pl.when(s1 > 1)
    def _():
      wait_sc(ec1, sc1)

    @pl.when(s1 % CH == 0)
    def _():
      load_chunk(s1)

    compute(s1, ec1)
    fire_sc(s1, ec1, sc1)
    return carry

  lax.fori_loop(0, (STEPS - 1) // 2, body, 0)

  wait_sc(ec0, sc0)
  compute(STEPS - 1, ec0)
  fire_sc(STEPS - 1, ec0, sc0)
  wait_sc(ec0, sc0)
  wait_sc(ec1, sc1)

  plsc.subcore_barrier()

  @pl.when(c == 0)
  def _():
    pltpu.sync_copy(eagg_sh.at[pl.ds(row0, STRIPE)],
                    eagg0_h.at[pl.ds(row0, STRIPE)])

  @pl.when(c == 1)
  def _():
    pltpu.sync_copy(eagg_sh.at[pl.ds(row0, STRIPE)],
                    eagg1_h.at[pl.ds(row0, STRIPE)])


def kernel(x, edge_index, edge_attr,
           W1q, b1q, W1k, b1k, W1v, b1v, W1e, W1s, b1s,
           W2q, b2q, W2k, b2k, W2v, b2v, W2e, W2s, b2s):
  srcw = edge_index[0].reshape(E // B, B)
  dstw = edge_index[1].reshape(E // B, B)
  zagg = jnp.zeros((N, D), jnp.float32)
  zea = jnp.zeros((N, 2 * DE), jnp.float32)

  def layer(qq, kt, vt):
    alpha, tmax = _sc_pass_a(srcw, dstw, edge_attr, qq, kt)
    a0, a1 = _sc_pass_b(srcw, dstw, alpha, tmax, vt, zagg)
    e0, e1 = _sc_pass_c(dstw, edge_attr, alpha, tmax, zea)
    return a0, a1, e0, e1

  r = lambda b: b.reshape(1, D)

  qq, kt, vt, sk1 = _tc_dense(x, W1q, r(b1q), W1k, r(b1k), W1v, r(b1v),
                              W1e.T, W1s, r(b1s))
  a0, a1, e0, e1 = layer(qq, kt, vt)
  qq2, kt2, vt2, sk2 = _tc_mid(a0, a1, e0, e1, sk1, W1e,
                               W2q, r(b2q), W2k, r(b2k), W2v, r(b2v),
                               W2e.T, W2s, r(b2s))
  b0, b1_, f0, f1 = layer(qq2, kt2, vt2)
  return _tc_fin(b0, b1_, f0, f1, sk2, W2e)


# speedup vs baseline: 1.2121x; 1.0164x over previous
import functools

import jax
import jax.numpy as jnp
from jax import lax
from jax.experimental import pallas as pl
from jax.experimental.pallas import tpu as pltpu
from jax.experimental.pallas import tpu_sc as plsc

N = 10000
E = 320000
D = 128
DE = 16
DQ = D + DE

NC = 2
NS = 16
NW = NC * NS
EW = E // NW
B = 80
STEPS = EW // B
TS = STEPS
CH = 5
CE = CH * B
STRIPE = N // NS
INV_SQRT_C = 1.0 / (128.0 ** 0.5)

_MESH = plsc.VectorSubcoreMesh(core_axis_name="c", subcore_axis_name="s")
_SC_PARAMS = pltpu.CompilerParams(needs_layout_passes=False,
                                  use_tc_tiling_on_sc=False)


def _project(xb, wq, bq, wet):
  q = (jnp.dot(xb, wq[...], preferred_element_type=jnp.float32)
       + bq[...]) * INV_SQRT_C
  qe = jnp.dot(q, wet[...], preferred_element_type=jnp.float32)
  return jnp.concatenate([q, qe], axis=1)


def _dense_body(x_ref, wq, bq, wk, bk, wv, bv, wet, ws, bs,
                qq_o, k_o, v_o, s_o):
  xb = x_ref[...]
  qq_o[...] = _project(xb, wq, bq, wet)
  k_o[...] = jnp.dot(xb, wk[...], preferred_element_type=jnp.float32) + bk[...]
  v_o[...] = jnp.dot(xb, wv[...], preferred_element_type=jnp.float32) + bv[...]
  s_o[...] = jnp.dot(xb, ws[...], preferred_element_type=jnp.float32) + bs[...]


def _combine(a0, a1, e0, e1, sk, we):
  es = e0[...] + e1[...]
  eagg = es[:, :DE]
  den = es[:, DE:DE + 1]
  num = a0[...] + a1[...] + jnp.dot(eagg, we[...],
                                    preferred_element_type=jnp.float32)
  agg = jnp.where(den > 0, num / den, 0.0)
  return agg + sk[...]


def _mid_body(a0, a1, e0, e1, sk, we,
              wq, bq, wk, bk, wv, bv, wet, ws, bs,
              qq_o, k_o, v_o, s_o):
  h = jnp.maximum(_combine(a0, a1, e0, e1, sk, we), 0.0)
  qq_o[...] = _project(h, wq, bq, wet)
  k_o[...] = jnp.dot(h, wk[...], preferred_element_type=jnp.float32) + bk[...]
  v_o[...] = jnp.dot(h, wv[...], preferred_element_type=jnp.float32) + bv[...]
  s_o[...] = jnp.dot(h, ws[...], preferred_element_type=jnp.float32) + bs[...]


def _fin_body(a0, a1, e0, e1, sk, we, out):
  out[...] = _combine(a0, a1, e0, e1, sk, we)


_R = 1000
_G = N // _R


def _row_spec(w):
  return pl.BlockSpec((_R, w), lambda i: (i, 0))


def _full_spec(shape):
  return pl.BlockSpec(shape, lambda i: tuple(0 for _ in shape))


_W128 = _full_spec((D, D))
_B128 = _full_spec((1, D))
_WET = _full_spec((D, DE))
_WE = _full_spec((DE, D))

_DENSE_OUT = [
    jax.ShapeDtypeStruct((N, DQ), jnp.float32),
    jax.ShapeDtypeStruct((N, D), jnp.float32),
    jax.ShapeDtypeStruct((N, D), jnp.float32),
    jax.ShapeDtypeStruct((N, D), jnp.float32),
]
_DENSE_OUT_SPECS = [_row_spec(DQ)] + [_row_spec(D)] * 3


def _tc_dense(x, wq, bq, wk, bk, wv, bv, wet, ws, bs):
  return pl.pallas_call(
      _dense_body,
      grid=(_G,),
      in_specs=[_row_spec(D), _W128, _B128, _W128, _B128, _W128, _B128,
                _WET, _W128, _B128],
      out_specs=_DENSE_OUT_SPECS,
      out_shape=_DENSE_OUT,
  )(x, wq, bq, wk, bk, wv, bv, wet, ws, bs)


def _tc_mid(a0, a1, e0, e1, sk, we, wq, bq, wk, bk, wv, bv, wet, ws, bs):
  return pl.pallas_call(
      _mid_body,
      grid=(_G,),
      in_specs=[_row_spec(D), _row_spec(D), _row_spec(2 * DE),
                _row_spec(2 * DE), _row_spec(D), _WE,
                _W128, _B128, _W128, _B128, _W128, _B128, _WET, _W128, _B128],
      out_specs=_DENSE_OUT_SPECS,
      out_shape=_DENSE_OUT,
  )(a0, a1, e0, e1, sk, we, wq, bq, wk, bk, wv, bv, wet, ws, bs)


def _tc_fin(a0, a1, e0, e1, sk, we):
  return pl.pallas_call(
      _fin_body,
      grid=(_G,),
      in_specs=[_row_spec(D), _row_spec(D), _row_spec(2 * DE),
                _row_spec(2 * DE), _row_spec(D), _WE],
      out_specs=_row_spec(D),
      out_shape=jax.ShapeDtypeStruct((N, D), jnp.float32),
  )(a0, a1, e0, e1, sk, we)


@functools.partial(
    pl.kernel,
    out_type=[
        jax.ShapeDtypeStruct((E,), jnp.float32),
        jax.ShapeDtypeStruct((NW * 16,), jnp.float32),
    ],
    mesh=_MESH,
    compiler_params=_SC_PARAMS,
    scratch_types=[
        pltpu.VMEM((TS, B), jnp.int32),
        pltpu.VMEM((TS, B), jnp.int32),
        pltpu.VMEM((CE, DE), jnp.float32),
        pltpu.VMEM((EW,), jnp.float32),
        pltpu.VMEM((B, DQ), jnp.float32),
        pltpu.VMEM((B, DQ), jnp.float32),
        pltpu.VMEM((B, D), jnp.float32),
        pltpu.VMEM((B, D), jnp.float32),
        pltpu.VMEM((16,), jnp.float32),
        pltpu.SemaphoreType.DMA,
        pltpu.SemaphoreType.DMA,
    ],
)
def _sc_pass_a(srcw_h, dstw_h, ea_h, qq_h, kt_h, alpha_h, tmax_h,
               srcw_c, dstw_c, ea_c, al_c, qq0, qq1, k0, k1, mx_v,
               g0, g1):
  c = lax.axis_index("c")
  s = lax.axis_index("s")
  wid = c * NS + s
  base = wid * EW
  trow = wid * TS
  lanes = lax.broadcasted_iota(jnp.int32, (16,), 0)

  pltpu.sync_copy(srcw_h.at[pl.ds(trow, TS)], srcw_c)
  pltpu.sync_copy(dstw_h.at[pl.ds(trow, TS)], dstw_c)
  pltpu.sync_copy(ea_h.at[pl.ds(base, CE)], ea_c)

  def fire(j, qq_b, k_b, sem):
    pltpu.make_async_copy(qq_h.at[dstw_c.at[j]], qq_b, sem).start()
    pltpu.make_async_copy(kt_h.at[srcw_c.at[j]], k_b, sem).start()

  def wait(qq_b, k_b, sem):
    pltpu.make_async_copy(qq_h.at[dstw_c.at[0]], qq_b, sem).wait()
    pltpu.make_async_copy(kt_h.at[srcw_c.at[0]], k_b, sem).wait()

  def load_chunk(j):
    pltpu.sync_copy(ea_h.at[pl.ds(base + (j // CH) * CE, CE)], ea_c)

  rotp = [(lanes + h) % 16 for h in (8, 4, 2, 1)]
  rotm = [(lanes - h) % 16 for h in (8, 4, 2, 1)]
  low = [((lanes // h) % 2) == 0 for h in (8, 4, 2, 1)]
  bitrev = (((lanes & 1) << 3) | ((lanes & 2) << 1)
            | ((lanes & 4) >> 1) | ((lanes & 8) >> 3))

  def _rot(v, t):
    return jnp.take_along_axis(v, t, axis=0, mode="promise_in_bounds")

  def _comb(x, y, hi):
    return jnp.where(low[hi], x + _rot(x, rotp[hi]), y + _rot(y, rotm[hi]))

  def compute(j, qq_b, k_b, mx):
    jr = (j % CH) * B

    def grp(g, mx):
      b0 = g * 16

      def acc(t):
        b = b0 + t
        a = qq_b[b, pl.ds(0, 16)] * k_b[b, pl.ds(0, 16)]
        for u in range(1, 4):
          a = a + qq_b[b, pl.ds(u * 16, 16)] * k_b[b, pl.ds(u * 16, 16)]
        c = qq_b[b, pl.ds(D, DE)] * ea_c[jr + b, :]
        for u in range(4, 8):
          c = c + qq_b[b, pl.ds(u * 16, 16)] * k_b[b, pl.ds(u * 16, 16)]
        return a + c

      def red(lo, size):
        if size == 1:
          return acc(lo)
        half = size // 2
        return _comb(red(lo, half), red(lo + half, half),
                     {2: 0, 4: 1, 8: 2, 16: 3}[size])

      av = _rot(red(0, 16), bitrev)
      al_c[pl.ds(j * B + b0, 16)] = av
      return jnp.maximum(mx, av)

    return lax.fori_loop(0, B // 16, grp, mx)

  fire(0, qq0, k0, g0)

  def body(k, mx):
    s0 = 2 * k
    s1 = 2 * k + 1
    fire(s1, qq1, k1, g1)

    @pl.when((s0 > 0) & (s0 % CH == 0))
    def _():
      load_chunk(s0)

    wait(qq0, k0, g0)
    mx = compute(s0, qq0, k0, mx)
    fire(s0 + 2, qq0, k0, g0)

    @pl.when(s1 % CH == 0)
    def _():
      load_chunk(s1)

    wait(qq1, k1, g1)
    mx = compute(s1, qq1, k1, mx)
    return mx

  mx = lax.fori_loop(0, (STEPS - 1) // 2, body,
                     jnp.full((16,), -jnp.inf, jnp.float32))
  wait(qq0, k0, g0)
  mx = compute(STEPS - 1, qq0, k0, mx)

  mx_v[...] = mx
  pltpu.sync_copy(mx_v, tmax_h.at[pl.ds(wid * 16, 16)])
  pltpu.sync_copy(al_c, alpha_h.at[pl.ds(base, EW)])


CHB = 25
CEB = CHB * B

@functools.partial(
    pl.kernel,
    out_type=[
        jax.ShapeDtypeStruct((N, D), jnp.float32),
        jax.ShapeDtypeStruct((N, D), jnp.float32),
    ],
    mesh=_MESH,
    compiler_params=_SC_PARAMS,
    scratch_types=[
        pltpu.VMEM((TS, B), jnp.int32),
        pltpu.VMEM((TS, B), jnp.int32),
        pltpu.VMEM((CEB,), jnp.float32),
        pltpu.VMEM((B, D), jnp.float32),
        pltpu.VMEM((B, D), jnp.float32),
        pltpu.VMEM((NW * 16,), jnp.float32),
        pltpu.VMEM_SHARED((N, D), jnp.float32),
        pltpu.SemaphoreType.DMA,
        pltpu.SemaphoreType.DMA,
        pltpu.SemaphoreType.DMA,
        pltpu.SemaphoreType.DMA,
    ],
)
def _sc_pass_b(srcw_h, dstw_h, alpha_h, tmax_h, vt_h, zagg_h,
               agg0_h, agg1_h,
               srcw_c, dstw_c, al_c, v0, v1,
               mxb_v, aggv_sh, gv0, gv1, sc0, sc1):
  c = lax.axis_index("c")
  s = lax.axis_index("s")
  wid = c * NS + s
  base = wid * EW
  trow = wid * TS
  row0 = s * STRIPE

  pltpu.sync_copy(zagg_h.at[pl.ds(row0, STRIPE)],
                  aggv_sh.at[pl.ds(row0, STRIPE)])
  plsc.subcore_barrier()

  pltpu.sync_copy(tmax_h, mxb_v)
  acc = mxb_v[pl.ds(0, 16)]
  for w in range(1, NW):
    acc = jnp.maximum(acc, mxb_v[pl.ds(w * 16, 16)])
  gmax = jnp.max(acc)

  pltpu.sync_copy(srcw_h.at[pl.ds(trow, TS)], srcw_c)
  pltpu.sync_copy(dstw_h.at[pl.ds(trow, TS)], dstw_c)
  pltpu.sync_copy(alpha_h.at[pl.ds(base, CEB)], al_c)

  def fire_v(j, v_b, sem):
    pltpu.make_async_copy(vt_h.at[srcw_c.at[j]], v_b, sem).start()

  def wait_v(v_b, sem):
    pltpu.make_async_copy(vt_h.at[srcw_c.at[0]], v_b, sem).wait()

  def fire_sc(j, v_b, sem):
    pltpu.async_copy(v_b, aggv_sh.at[dstw_c.at[j]], sem, add=True)

  def wait_sc(v_b, sem):
    pltpu.make_async_copy(v_b, aggv_sh.at[dstw_c.at[0]], sem).wait()

  def load_chunk(j):
    pltpu.sync_copy(alpha_h.at[pl.ds(base + (j // CHB) * CEB, CEB)], al_c)

  def compute(j, v_b):
    jr = (j % CHB) * B
    for g in range(B // 16):
      exg = jnp.exp(al_c[pl.ds(jr + g * 16, 16)] - gmax)
      for t in range(16):
        b = g * 16 + t
        sx = jnp.take_along_axis(exg, jnp.full((16,), t, jnp.int32),
                                 axis=0, mode="promise_in_bounds")
        for u in range(D // 16):
          v_b[b, pl.ds(u * 16, 16)] = v_b[b, pl.ds(u * 16, 16)] * sx

  fire_v(0, v0, gv0)
  fire_v(1, v1, gv1)

  def body(k, carry):
    s0 = 2 * k
    s1 = 2 * k + 1

    @pl.when((s0 > 0) & (s0 % CHB == 0))
    def _():
      load_chunk(s0)

    wait_v(v0, gv0)
    compute(s0, v0)
    fire_sc(s0, v0, sc0)
    wait_sc(v0, sc0)
    fire_v(jnp.minimum(s0 + 2, STEPS - 1), v0, gv0)

    @pl.when(s1 % CHB == 0)
    def _():
      load_chunk(s1)

    wait_v(v1, gv1)
    compute(s1, v1)
    fire_sc(s1, v1, sc1)
    wait_sc(v1, sc1)
    fire_v(jnp.minimum(s1 + 2, STEPS - 1), v1, gv1)
    return carry

  lax.fori_loop(0, (STEPS - 1) // 2, body, 0)

  wait_v(v0, gv0)
  compute(STEPS - 1, v0)
  fire_sc(STEPS - 1, v0, sc0)
  wait_sc(v0, sc0)
  wait_v(v1, gv1)

  plsc.subcore_barrier()

  @pl.when(c == 0)
  def _():
    pltpu.sync_copy(aggv_sh.at[pl.ds(row0, STRIPE)],
                    agg0_h.at[pl.ds(row0, STRIPE)])

  @pl.when(c == 1)
  def _():
    pltpu.sync_copy(aggv_sh.at[pl.ds(row0, STRIPE)],
                    agg1_h.at[pl.ds(row0, STRIPE)])


@functools.partial(
    pl.kernel,
    out_type=[
        jax.ShapeDtypeStruct((N, 2 * DE), jnp.float32),
        jax.ShapeDtypeStruct((N, 2 * DE), jnp.float32),
    ],
    mesh=_MESH,
    compiler_params=_SC_PARAMS,
    scratch_types=[
        pltpu.VMEM((TS, B), jnp.int32),
        pltpu.VMEM((CE, DE), jnp.float32),
        pltpu.VMEM((EW,), jnp.float32),
        pltpu.VMEM((B, 2 * DE), jnp.float32),
        pltpu.VMEM((B, 2 * DE), jnp.float32),
        pltpu.VMEM((NW * 16,), jnp.float32),
        pltpu.VMEM_SHARED((N, 2 * DE), jnp.float32),
        pltpu.SemaphoreType.DMA,
        pltpu.SemaphoreType.DMA,
    ],
)
def _sc_pass_c(dstw_h, ea_h, alpha_h, tmax_h, zea_h,
               eagg0_h, eagg1_h,
               dstw_c, ea_c, al_c, ec0, ec1, mxb_v, eagg_sh, sc0, sc1):
  c = lax.axis_index("c")
  s = lax.axis_index("s")
  wid = c * NS + s
  base = wid * EW
  trow = wid * TS
  lanes = lax.broadcasted_iota(jnp.int32, (16,), 0)
  row0 = s * STRIPE

  pltpu.sync_copy(zea_h.at[pl.ds(row0, STRIPE)],
                  eagg_sh.at[pl.ds(row0, STRIPE)])
  plsc.subcore_barrier()

  pltpu.sync_copy(tmax_h, mxb_v)
  acc = mxb_v[pl.ds(0, 16)]
  for w in range(1, NW):
    acc = jnp.maximum(acc, mxb_v[pl.ds(w * 16, 16)])
  gmax = jnp.max(acc)

  pltpu.sync_copy(dstw_h.at[pl.ds(trow, TS)], dstw_c)
  pltpu.sync_copy(alpha_h.at[pl.ds(base, EW)], al_c)
  pltpu.sync_copy(ea_h.at[pl.ds(base, CE)], ea_c)

  def fire_sc(j, ec, sem):
    pltpu.async_copy(ec, eagg_sh.at[dstw_c.at[j]], sem, add=True)

  def wait_sc(ec, sem):
    pltpu.make_async_copy(ec, eagg_sh.at[dstw_c.at[0]], sem).wait()

  def load_chunk(j):
    pltpu.sync_copy(ea_h.at[pl.ds(base + (j // CH) * CE, CE)], ea_c)

  def compute(j, ec):
    jr = (j % CH) * B
    for g in range(B // 16):
      exg = jnp.exp(al_c[pl.ds(j * B + g * 16, 16)] - gmax)
      for t in range(16):
        b = g * 16 + t
        sx = jnp.take_along_axis(exg, jnp.full((16,), t, jnp.int32),
                                 axis=0, mode="promise_in_bounds")
        ec[b, pl.ds(0, DE)] = ea_c[jr + b, :] * sx
        ec[b, pl.ds(DE, DE)] = jnp.where(lanes == 0, sx, 0.0)

  def body(k, carry):
    s0 = 2 * k
    s1 = 2 * k + 1

    @pl.when(s0 > 0)
    def _():
      wait_sc(ec0, sc0)

    @pl.when((s0 > 0) & (s0 % CH == 0))
    def _():
      load_chunk(s0)

    compute(s0, ec0)
    fire_sc(s0, ec0, sc0)

    @pl.when(s1 > 1)
    def _():
      wait_sc(ec1, sc1)

    @pl.when(s1 % CH == 0)
    def _():
      load_chunk(s1)

    compute(s1, ec1)
    fire_sc(s1, ec1, sc1)
    return carry

  lax.fori_loop(0, (STEPS - 1) // 2, body, 0)

  wait_sc(ec0, sc0)
  compute(STEPS - 1, ec0)
  fire_sc(STEPS - 1, ec0, sc0)
  wait_sc(ec0, sc0)
  wait_sc(ec1, sc1)

  plsc.subcore_barrier()

  @pl.when(c == 0)
  def _():
    pltpu.sync_copy(eagg_sh.at[pl.ds(row0, STRIPE)],
                    eagg0_h.at[pl.ds(row0, STRIPE)])

  @pl.when(c == 1)
  def _():
    pltpu.sync_copy(eagg_sh.at[pl.ds(row0, STRIPE)],
                    eagg1_h.at[pl.ds(row0, STRIPE)])


def kernel(x, edge_index, edge_attr,
           W1q, b1q, W1k, b1k, W1v, b1v, W1e, W1s, b1s,
           W2q, b2q, W2k, b2k, W2v, b2v, W2e, W2s, b2s):
  srcw = edge_index[0].reshape(E // B, B)
  dstw = edge_index[1].reshape(E // B, B)
  zagg = jnp.zeros((N, D), jnp.float32)
  zea = jnp.zeros((N, 2 * DE), jnp.float32)

  def layer(qq, kt, vt):
    alpha, tmax = _sc_pass_a(srcw, dstw, edge_attr, qq, kt)
    a0, a1 = _sc_pass_b(srcw, dstw, alpha, tmax, vt, zagg)
    e0, e1 = _sc_pass_c(dstw, edge_attr, alpha, tmax, zea)
    return a0, a1, e0, e1

  r = lambda b: b.reshape(1, D)

  qq, kt, vt, sk1 = _tc_dense(x, W1q, r(b1q), W1k, r(b1k), W1v, r(b1v),
                              W1e.T, W1s, r(b1s))
  a0, a1, e0, e1 = layer(qq, kt, vt)
  qq2, kt2, vt2, sk2 = _tc_mid(a0, a1, e0, e1, sk1, W1e,
                               W2q, r(b2q), W2k, r(b2k), W2v, r(b2v),
                               W2e.T, W2s, r(b2s))
  b0, b1_, f0, f1 = layer(qq2, kt2, vt2)
  return _tc_fin(b0, b1_, f0, f1, sk2, W2e)


# speedup vs baseline: 1.4477x; 1.1944x over previous
import functools

import jax
import jax.numpy as jnp
from jax import lax
from jax.experimental import pallas as pl
from jax.experimental.pallas import tpu as pltpu
from jax.experimental.pallas import tpu_sc as plsc

N = 10000
E = 320000
D = 128
DE = 16
DQ = D + 2 * DE

NC = 2
NS = 16
NW = NC * NS
EW = E // NW
B = 80
STEPS = EW // B
TS = STEPS
CH = 5
CE = CH * B
STRIPE = N // NS
INV_SQRT_C = 1.0 / (128.0 ** 0.5)

_MESH = plsc.VectorSubcoreMesh(core_axis_name="c", subcore_axis_name="s")
_SC_PARAMS = pltpu.CompilerParams(needs_layout_passes=False,
                                  use_tc_tiling_on_sc=False)


def _project(xb, wq, bq, wet):
  q = (jnp.dot(xb, wq[...], preferred_element_type=jnp.float32)
       + bq[...]) * INV_SQRT_C
  qe = jnp.dot(q, wet[...], preferred_element_type=jnp.float32)
  return jnp.concatenate([q, qe, jnp.zeros_like(qe)],
                         axis=1).astype(jnp.bfloat16)


def _dense_body(x_ref, wq, bq, wk, bk, wv, bv, wet, ws, bs,
                qq_o, k_o, v_o, s_o):
  xb = x_ref[...]
  qq_o[...] = _project(xb, wq, bq, wet)
  k_o[...] = (jnp.dot(xb, wk[...], preferred_element_type=jnp.float32)
              + bk[...]).astype(jnp.bfloat16)
  v_o[...] = jnp.dot(xb, wv[...], preferred_element_type=jnp.float32) + bv[...]
  s_o[...] = jnp.dot(xb, ws[...], preferred_element_type=jnp.float32) + bs[...]


def _combine(a0, a1, e0, e1, sk, we):
  es = e0[...] + e1[...]
  eagg = es[:, :DE]
  den = es[:, DE:DE + 1]
  num = a0[...] + a1[...] + jnp.dot(eagg, we[...],
                                    preferred_element_type=jnp.float32)
  agg = jnp.where(den > 0, num / den, 0.0)
  return agg + sk[...]


def _mid_body(a0, a1, e0, e1, sk, we,
              wq, bq, wk, bk, wv, bv, wet, ws, bs,
              qq_o, k_o, v_o, s_o):
  h = jnp.maximum(_combine(a0, a1, e0, e1, sk, we), 0.0)
  qq_o[...] = _project(h, wq, bq, wet)
  k_o[...] = (jnp.dot(h, wk[...], preferred_element_type=jnp.float32)
              + bk[...]).astype(jnp.bfloat16)
  v_o[...] = jnp.dot(h, wv[...], preferred_element_type=jnp.float32) + bv[...]
  s_o[...] = jnp.dot(h, ws[...], preferred_element_type=jnp.float32) + bs[...]


def _fin_body(a0, a1, e0, e1, sk, we, out):
  out[...] = _combine(a0, a1, e0, e1, sk, we)


_R = 1000
_G = N // _R


def _row_spec(w):
  return pl.BlockSpec((_R, w), lambda i: (i, 0))


def _full_spec(shape):
  return pl.BlockSpec(shape, lambda i: tuple(0 for _ in shape))


_W128 = _full_spec((D, D))
_B128 = _full_spec((1, D))
_WET = _full_spec((D, DE))
_WE = _full_spec((DE, D))

_DENSE_OUT = [
    jax.ShapeDtypeStruct((N, DQ), jnp.bfloat16),
    jax.ShapeDtypeStruct((N, D), jnp.bfloat16),
    jax.ShapeDtypeStruct((N, D), jnp.float32),
    jax.ShapeDtypeStruct((N, D), jnp.float32),
]
_DENSE_OUT_SPECS = [_row_spec(DQ)] + [_row_spec(D)] * 3


def _tc_dense(x, wq, bq, wk, bk, wv, bv, wet, ws, bs):
  return pl.pallas_call(
      _dense_body,
      grid=(_G,),
      in_specs=[_row_spec(D), _W128, _B128, _W128, _B128, _W128, _B128,
                _WET, _W128, _B128],
      out_specs=_DENSE_OUT_SPECS,
      out_shape=_DENSE_OUT,
  )(x, wq, bq, wk, bk, wv, bv, wet, ws, bs)


def _tc_mid(a0, a1, e0, e1, sk, we, wq, bq, wk, bk, wv, bv, wet, ws, bs):
  return pl.pallas_call(
      _mid_body,
      grid=(_G,),
      in_specs=[_row_spec(D), _row_spec(D), _row_spec(2 * DE),
                _row_spec(2 * DE), _row_spec(D), _WE,
                _W128, _B128, _W128, _B128, _W128, _B128, _WET, _W128, _B128],
      out_specs=_DENSE_OUT_SPECS,
      out_shape=_DENSE_OUT,
  )(a0, a1, e0, e1, sk, we, wq, bq, wk, bk, wv, bv, wet, ws, bs)


def _tc_fin(a0, a1, e0, e1, sk, we):
  return pl.pallas_call(
      _fin_body,
      grid=(_G,),
      in_specs=[_row_spec(D), _row_spec(D), _row_spec(2 * DE),
                _row_spec(2 * DE), _row_spec(D), _WE],
      out_specs=_row_spec(D),
      out_shape=jax.ShapeDtypeStruct((N, D), jnp.float32),
  )(a0, a1, e0, e1, sk, we)


@functools.partial(
    pl.kernel,
    out_type=[
        jax.ShapeDtypeStruct((E,), jnp.float32),
        jax.ShapeDtypeStruct((NW * 16,), jnp.float32),
    ],
    mesh=_MESH,
    compiler_params=_SC_PARAMS,
    scratch_types=[
        pltpu.VMEM((TS, B), jnp.int32),
        pltpu.VMEM((TS, B), jnp.int32),
        pltpu.VMEM((CE * DE,), jnp.bfloat16),
        pltpu.VMEM((EW,), jnp.float32),
        pltpu.VMEM((B, DQ), jnp.bfloat16),
        pltpu.VMEM((B, DQ), jnp.bfloat16),
        pltpu.VMEM((B, D), jnp.bfloat16),
        pltpu.VMEM((B, D), jnp.bfloat16),
        pltpu.VMEM((16,), jnp.float32),
        pltpu.SemaphoreType.DMA,
        pltpu.SemaphoreType.DMA,
    ],
)
def _sc_pass_a(srcw_h, dstw_h, ea_h, qq_h, kt_h, alpha_h, tmax_h,
               srcw_c, dstw_c, ea_c, al_c, qq0, qq1, k0, k1, mx_v,
               g0, g1):
  c = lax.axis_index("c")
  s = lax.axis_index("s")
  wid = c * NS + s
  base = wid * EW
  trow = wid * TS
  lanes = lax.broadcasted_iota(jnp.int32, (16,), 0)

  pltpu.sync_copy(srcw_h.at[pl.ds(trow, TS)], srcw_c)
  pltpu.sync_copy(dstw_h.at[pl.ds(trow, TS)], dstw_c)
  pltpu.sync_copy(ea_h.at[pl.ds(base * DE, CE * DE)], ea_c)

  def fire(j, qq_b, k_b, sem):
    pltpu.make_async_copy(qq_h.at[dstw_c.at[j]], qq_b, sem).start()
    pltpu.make_async_copy(kt_h.at[srcw_c.at[j]], k_b, sem).start()

  def wait(qq_b, k_b, sem):
    pltpu.make_async_copy(qq_h.at[dstw_c.at[0]], qq_b, sem).wait()
    pltpu.make_async_copy(kt_h.at[srcw_c.at[0]], k_b, sem).wait()

  def load_chunk(j):
    pltpu.sync_copy(
        ea_h.at[pl.ds((base + (j // CH) * CE) * DE, CE * DE)], ea_c)

  rotp = [(lanes + h) % 16 for h in (8, 4, 2, 1)]
  rotm = [(lanes - h) % 16 for h in (8, 4, 2, 1)]
  low = [((lanes // h) % 2) == 0 for h in (8, 4, 2, 1)]
  bitrev = (((lanes & 1) << 3) | ((lanes & 2) << 1)
            | ((lanes & 4) >> 1) | ((lanes & 8) >> 3))

  def _rot(v, t):
    return jnp.take_along_axis(v, t, axis=0, mode="promise_in_bounds")

  def _comb(x, y, hi):
    return jnp.where(low[hi], x + _rot(x, rotp[hi]), y + _rot(y, rotm[hi]))

  UNP = plsc.PackFormat.INTERLEAVED

  def compute(j, qq_b, k_b, mx):
    jrf = (j % CH) * B * DE

    def grp(g, mx):
      b0 = g * 16

      def qk(b, c):
        a = None
        for u in range(4):
          xa, xb = plsc.unpack(qq_b[b, pl.ds(u * 32, 32)], format=UNP)
          ya, yb = plsc.unpack(k_b[b, pl.ds(u * 32, 32)], format=UNP)
          a = xa * ya if a is None else a + xa * ya
          c = c + xb * yb
        return a + c

      def red2(lo):
        ea32 = ea_c[pl.ds(jrf + (b0 + lo) * DE, 32)]
        eaA, eaB = plsc.unpack(ea32, format=UNP)
        qa0, qb0 = plsc.unpack(qq_b[b0 + lo, pl.ds(D, 32)], format=UNP)
        acc_e = qk(b0 + lo, qa0 * eaA + qb0 * eaB)
        qa1, qb1 = plsc.unpack(qq_b[b0 + lo + 1, pl.ds(D, 32)], format=UNP)
        acc_o = qk(b0 + lo + 1,
                   qa1 * _rot(eaA, rotp[0]) + qb1 * _rot(eaB, rotp[0]))
        return _comb(acc_e, acc_o, 0)

      def red(lo, size):
        if size == 2:
          return red2(lo)
        half = size // 2
        return _comb(red(lo, half), red(lo + half, half),
                     {4: 1, 8: 2, 16: 3}[size])

      av = _rot(red(0, 16), bitrev)
      al_c[pl.ds(j * B + b0, 16)] = av
      return jnp.maximum(mx, av)

    return lax.fori_loop(0, B // 16, grp, mx)

  fire(0, qq0, k0, g0)

  def body(k, mx):
    s0 = 2 * k
    s1 = 2 * k + 1
    fire(s1, qq1, k1, g1)

    @pl.when((s0 > 0) & (s0 % CH == 0))
    def _():
      load_chunk(s0)

    wait(qq0, k0, g0)
    mx = compute(s0, qq0, k0, mx)
    fire(s0 + 2, qq0, k0, g0)

    @pl.when(s1 % CH == 0)
    def _():
      load_chunk(s1)

    wait(qq1, k1, g1)
    mx = compute(s1, qq1, k1, mx)
    return mx

  mx = lax.fori_loop(0, (STEPS - 1) // 2, body,
                     jnp.full((16,), -jnp.inf, jnp.float32))
  wait(qq0, k0, g0)
  mx = compute(STEPS - 1, qq0, k0, mx)

  mx_v[...] = mx
  pltpu.sync_copy(mx_v, tmax_h.at[pl.ds(wid * 16, 16)])
  pltpu.sync_copy(al_c, alpha_h.at[pl.ds(base, EW)])


CHB = 25
CEB = CHB * B

@functools.partial(
    pl.kernel,
    out_type=[
        jax.ShapeDtypeStruct((N, D), jnp.float32),
        jax.ShapeDtypeStruct((N, D), jnp.float32),
    ],
    mesh=_MESH,
    compiler_params=_SC_PARAMS,
    scratch_types=[
        pltpu.VMEM((TS, B), jnp.int32),
        pltpu.VMEM((TS, B), jnp.int32),
        pltpu.VMEM((CEB,), jnp.float32),
        pltpu.VMEM((B, D), jnp.float32),
        pltpu.VMEM((B, D), jnp.float32),
        pltpu.VMEM((NW * 16,), jnp.float32),
        pltpu.VMEM_SHARED((N, D), jnp.float32),
        pltpu.SemaphoreType.DMA,
        pltpu.SemaphoreType.DMA,
        pltpu.SemaphoreType.DMA,
        pltpu.SemaphoreType.DMA,
    ],
)
def _sc_pass_b(srcw_h, dstw_h, alpha_h, tmax_h, vt_h, zagg_h,
               agg0_h, agg1_h,
               srcw_c, dstw_c, al_c, v0, v1,
               mxb_v, aggv_sh, gv0, gv1, sc0, sc1):
  c = lax.axis_index("c")
  s = lax.axis_index("s")
  wid = c * NS + s
  base = wid * EW
  trow = wid * TS
  row0 = s * STRIPE

  pltpu.sync_copy(zagg_h.at[pl.ds(row0, STRIPE)],
                  aggv_sh.at[pl.ds(row0, STRIPE)])
  plsc.subcore_barrier()

  pltpu.sync_copy(tmax_h, mxb_v)
  acc = mxb_v[pl.ds(0, 16)]
  for w in range(1, NW):
    acc = jnp.maximum(acc, mxb_v[pl.ds(w * 16, 16)])
  gmax = jnp.max(acc)

  pltpu.sync_copy(srcw_h.at[pl.ds(trow, TS)], srcw_c)
  pltpu.sync_copy(dstw_h.at[pl.ds(trow, TS)], dstw_c)
  pltpu.sync_copy(alpha_h.at[pl.ds(base, CEB)], al_c)

  def fire_v(j, v_b, sem):
    pltpu.make_async_copy(vt_h.at[srcw_c.at[j]], v_b, sem).start()

  def wait_v(v_b, sem):
    pltpu.make_async_copy(vt_h.at[srcw_c.at[0]], v_b, sem).wait()

  def fire_sc(j, v_b, sem):
    pltpu.async_copy(v_b, aggv_sh.at[dstw_c.at[j]], sem, add=True)

  def wait_sc(v_b, sem):
    pltpu.make_async_copy(v_b, aggv_sh.at[dstw_c.at[0]], sem).wait()

  def load_chunk(j):
    pltpu.sync_copy(alpha_h.at[pl.ds(base + (j // CHB) * CEB, CEB)], al_c)

  def compute(j, v_b):
    jr = (j % CHB) * B
    for g in range(B // 16):
      exg = jnp.exp(al_c[pl.ds(jr + g * 16, 16)] - gmax)
      for t in range(16):
        b = g * 16 + t
        sx = jnp.take_along_axis(exg, jnp.full((16,), t, jnp.int32),
                                 axis=0, mode="promise_in_bounds")
        for u in range(D // 16):
          v_b[b, pl.ds(u * 16, 16)] = v_b[b, pl.ds(u * 16, 16)] * sx

  fire_v(0, v0, gv0)
  fire_v(1, v1, gv1)

  def body(k, carry):
    s0 = 2 * k
    s1 = 2 * k + 1

    @pl.when((s0 > 0) & (s0 % CHB == 0))
    def _():
      load_chunk(s0)

    wait_v(v0, gv0)
    compute(s0, v0)
    fire_sc(s0, v0, sc0)
    wait_sc(v0, sc0)
    fire_v(jnp.minimum(s0 + 2, STEPS - 1), v0, gv0)

    @pl.when(s1 % CHB == 0)
    def _():
      load_chunk(s1)

    wait_v(v1, gv1)
    compute(s1, v1)
    fire_sc(s1, v1, sc1)
    wait_sc(v1, sc1)
    fire_v(jnp.minimum(s1 + 2, STEPS - 1), v1, gv1)
    return carry

  lax.fori_loop(0, (STEPS - 1) // 2, body, 0)

  wait_v(v0, gv0)
  compute(STEPS - 1, v0)
  fire_sc(STEPS - 1, v0, sc0)
  wait_sc(v0, sc0)
  wait_v(v1, gv1)

  plsc.subcore_barrier()

  @pl.when(c == 0)
  def _():
    pltpu.sync_copy(aggv_sh.at[pl.ds(row0, STRIPE)],
                    agg0_h.at[pl.ds(row0, STRIPE)])

  @pl.when(c == 1)
  def _():
    pltpu.sync_copy(aggv_sh.at[pl.ds(row0, STRIPE)],
                    agg1_h.at[pl.ds(row0, STRIPE)])


@functools.partial(
    pl.kernel,
    out_type=[
        jax.ShapeDtypeStruct((N, 2 * DE), jnp.float32),
        jax.ShapeDtypeStruct((N, 2 * DE), jnp.float32),
    ],
    mesh=_MESH,
    compiler_params=_SC_PARAMS,
    scratch_types=[
        pltpu.VMEM((TS, B), jnp.int32),
        pltpu.VMEM((CE, DE), jnp.float32),
        pltpu.VMEM((EW,), jnp.float32),
        pltpu.VMEM((B, 2 * DE), jnp.float32),
        pltpu.VMEM((B, 2 * DE), jnp.float32),
        pltpu.VMEM((NW * 16,), jnp.float32),
        pltpu.VMEM_SHARED((N, 2 * DE), jnp.float32),
        pltpu.SemaphoreType.DMA,
        pltpu.SemaphoreType.DMA,
    ],
)
def _sc_pass_c(dstw_h, ea_h, alpha_h, tmax_h, zea_h,
               eagg0_h, eagg1_h,
               dstw_c, ea_c, al_c, ec0, ec1, mxb_v, eagg_sh, sc0, sc1):
  c = lax.axis_index("c")
  s = lax.axis_index("s")
  wid = c * NS + s
  base = wid * EW
  trow = wid * TS
  lanes = lax.broadcasted_iota(jnp.int32, (16,), 0)
  row0 = s * STRIPE

  pltpu.sync_copy(zea_h.at[pl.ds(row0, STRIPE)],
                  eagg_sh.at[pl.ds(row0, STRIPE)])
  plsc.subcore_barrier()

  pltpu.sync_copy(tmax_h, mxb_v)
  acc = mxb_v[pl.ds(0, 16)]
  for w in range(1, NW):
    acc = jnp.maximum(acc, mxb_v[pl.ds(w * 16, 16)])
  gmax = jnp.max(acc)

  pltpu.sync_copy(dstw_h.at[pl.ds(trow, TS)], dstw_c)
  pltpu.sync_copy(alpha_h.at[pl.ds(base, EW)], al_c)
  pltpu.sync_copy(ea_h.at[pl.ds(base, CE)], ea_c)

  def fire_sc(j, ec, sem):
    pltpu.async_copy(ec, eagg_sh.at[dstw_c.at[j]], sem, add=True)

  def wait_sc(ec, sem):
    pltpu.make_async_copy(ec, eagg_sh.at[dstw_c.at[0]], sem).wait()

  def load_chunk(j):
    pltpu.sync_copy(ea_h.at[pl.ds(base + (j // CH) * CE, CE)], ea_c)

  def compute(j, ec):
    jr = (j % CH) * B
    for g in range(B // 16):
      exg = jnp.exp(al_c[pl.ds(j * B + g * 16, 16)] - gmax)
      for t in range(16):
        b = g * 16 + t
        sx = jnp.take_along_axis(exg, jnp.full((16,), t, jnp.int32),
                                 axis=0, mode="promise_in_bounds")
        ec[b, pl.ds(0, DE)] = ea_c[jr + b, :] * sx
        ec[b, pl.ds(DE, DE)] = jnp.where(lanes == 0, sx, 0.0)

  def body(k, carry):
    s0 = 2 * k
    s1 = 2 * k + 1

    @pl.when(s0 > 0)
    def _():
      wait_sc(ec0, sc0)

    @pl.when((s0 > 0) & (s0 % CH == 0))
    def _():
      load_chunk(s0)

    compute(s0, ec0)
    fire_sc(s0, ec0, sc0)

    @pl.when(s1 > 1)
    def _():
      wait_sc(ec1, sc1)

    @pl.when(s1 % CH == 0)
    def _():
      load_chunk(s1)

    compute(s1, ec1)
    fire_sc(s1, ec1, sc1)
    return carry

  lax.fori_loop(0, (STEPS - 1) // 2, body, 0)

  wait_sc(ec0, sc0)
  compute(STEPS - 1, ec0)
  fire_sc(STEPS - 1, ec0, sc0)
  wait_sc(ec0, sc0)
  wait_sc(ec1, sc1)

  plsc.subcore_barrier()

  @pl.when(c == 0)
  def _():
    pltpu.sync_copy(eagg_sh.at[pl.ds(row0, STRIPE)],
                    eagg0_h.at[pl.ds(row0, STRIPE)])

  @pl.when(c == 1)
  def _():
    pltpu.sync_copy(eagg_sh.at[pl.ds(row0, STRIPE)],
                    eagg1_h.at[pl.ds(row0, STRIPE)])


def kernel(x, edge_index, edge_attr,
           W1q, b1q, W1k, b1k, W1v, b1v, W1e, W1s, b1s,
           W2q, b2q, W2k, b2k, W2v, b2v, W2e, W2s, b2s):
  srcw = edge_index[0].reshape(E // B, B)
  dstw = edge_index[1].reshape(E // B, B)
  zagg = jnp.zeros((N, D), jnp.float32)
  zea = jnp.zeros((N, 2 * DE), jnp.float32)

  eah = edge_attr.astype(jnp.bfloat16).reshape(E * DE)

  def layer(qq, kt, vt):
    alpha, tmax = _sc_pass_a(srcw, dstw, eah, qq, kt)
    a0, a1 = _sc_pass_b(srcw, dstw, alpha, tmax, vt, zagg)
    e0, e1 = _sc_pass_c(dstw, edge_attr, alpha, tmax, zea)
    return a0, a1, e0, e1

  r = lambda b: b.reshape(1, D)

  qq, kt, vt, sk1 = _tc_dense(x, W1q, r(b1q), W1k, r(b1k), W1v, r(b1v),
                              W1e.T, W1s, r(b1s))
  a0, a1, e0, e1 = layer(qq, kt, vt)
  qq2, kt2, vt2, sk2 = _tc_mid(a0, a1, e0, e1, sk1, W1e,
                               W2q, r(b2q), W2k, r(b2k), W2v, r(b2v),
                               W2e.T, W2s, r(b2s))
  b0, b1_, f0, f1 = layer(qq2, kt2, vt2)
  return _tc_fin(b0, b1_, f0, f1, sk2, W2e)
